# Initial kernel scaffold; baseline (speedup 1.0000x reference)
import jax
import jax.numpy as jnp
from jax.experimental import pallas as pl


def kernel(encoder_out, decoder_in, masks_mat, dec_embed_weight, ooKB_weight, Wqa, bqa, Wqs, bqs, Wk, bk, Wd, bd):
    raise NotImplementedError("write your pallas kernel here")


# speedup vs baseline: 29.1325x; 29.1325x over previous
import math
import jax
import jax.numpy as jnp
from jax.experimental import pallas as pl

_UB = 8


def _prologue(enc_ref, wqa_ref, bqa_ref, embs_ref, wqs_ref, bqs_ref,
              wk_ref, bk_ref, dec_ref, qac_ref, qse_ref, kv_ref):
    V1, Dh = embs_ref.shape
    V = V1 - 1
    A = wk_ref.shape[1]
    BU = dec_ref.shape[0]
    embs = embs_ref[...]
    qac_ref[...] = (jnp.dot(enc_ref[...], wqa_ref[...],
                            preferred_element_type=jnp.float32)
                    + bqa_ref[...])
    kv_ref[:, :A] = (jnp.dot(embs, wk_ref[...],
                             preferred_element_type=jnp.float32)
                     + bk_ref[...])
    kv_ref[:, A:] = embs
    onehot = (dec_ref[...] ==
              jax.lax.broadcasted_iota(jnp.int32, (BU, V), 1)
              ).astype(jnp.float32)
    semantic = jnp.dot(onehot, embs[:V, :], preferred_element_type=jnp.float32)
    qse_ref[...] = (jnp.dot(semantic, wqs_ref[...],
                            preferred_element_type=jnp.float32)
                    + bqs_ref[...])


def _main(qac_ref, qse_ref, idxr_ref, idxc_ref, kv_ref, wd_ref, bd_ref,
          ptr_ref, h_ref, db_ref):
    C = idxr_ref.shape[2]
    V1 = kv_ref.shape[0]
    A = qac_ref.shape[2]
    inv_sqrt_a = 1.0 / math.sqrt(A)
    qac = qac_ref[0]
    kv = kv_ref[...]
    wd = wd_ref[...]
    bd = bd_ref[...]
    iota_v = jax.lax.broadcasted_iota(jnp.int32, (C, V1), 1)
    iota_r = jax.lax.broadcasted_iota(jnp.int32, (C, C), 0)
    iota_c = jax.lax.broadcasted_iota(jnp.int32, (C, C), 1)
    for j in range(_UB):
        idx_col = idxc_ref[0, j]
        idx_row = idxr_ref[0, j:j + 1, :]
        G = (idx_col == iota_v).astype(jnp.float32)
        eq = idx_col == idx_row
        has_later = jnp.any(eq & (iota_c > iota_r), axis=1,
                            keepdims=True)
        S = jnp.where(has_later, 0.0, G)
        kvg = jnp.dot(G, kv, preferred_element_type=jnp.float32)
        keys = kvg[:, :A]
        values = kvg[:, A:]
        q = qac + qse_ref[0, j:j + 1, :]
        logits = jax.lax.dot_general(
            q, keys, (((1,), (1,)), ((), ())),
            preferred_element_type=jnp.float32) * inv_sqrt_a
        logits = jnp.where(idx_row < 0, -1.0e9, logits)
        m = jnp.max(logits, axis=1, keepdims=True)
        e = jnp.exp(logits - m)
        atten = e / jnp.sum(e, axis=1, keepdims=True)
        x = jnp.dot(atten, values, preferred_element_type=jnp.float32)
        h_ref[0, :, j, :] = x
        db_ref[0, :, j, :] = jnp.dot(x, wd,
                                     preferred_element_type=jnp.float32) + bd
        ptr_ref[0, :, j, :] = jnp.dot(atten, S,
                                      preferred_element_type=jnp.float32)


def kernel(encoder_out, decoder_in, masks_mat, dec_embed_weight, ooKB_weight,
           Wqa, bqa, Wqs, bqs, Wk, bk, Wd, bd):
    B, T, Eh = encoder_out.shape
    U = decoder_in.shape[1]
    C = masks_mat.shape[2]
    V, Dh = dec_embed_weight.shape
    A = Wk.shape[1]
    J = Wd.shape[1]
    V1 = V + 1
    f32 = jnp.float32

    embs = jnp.concatenate([dec_embed_weight, ooKB_weight], axis=0)
    enc2d = encoder_out.reshape(B * T, Eh)
    dec2d = decoder_in.reshape(B * U, 1).astype(jnp.int32)

    qac2d, qse2d, kv = pl.pallas_call(
        _prologue,
        out_shape=(
            jax.ShapeDtypeStruct((B * T, A), f32),
            jax.ShapeDtypeStruct((B * U, A), f32),
            jax.ShapeDtypeStruct((V1, A + Dh), f32),
        ),
    )(enc2d, Wqa, bqa.reshape(1, A), embs, Wqs, bqs.reshape(1, A),
      Wk, bk.reshape(1, A), dec2d)

    qac3 = qac2d.reshape(B, T, A)
    qse3 = qse2d.reshape(B, U, A)
    masks_row = masks_mat.astype(jnp.int32)
    masks_col = masks_row[..., None]

    nu = pl.cdiv(U, _UB)
    grid = (B, nu)
    ptr, h_ptr, dbias = pl.pallas_call(
        _main,
        grid=grid,
        in_specs=[
            pl.BlockSpec((1, T, A), lambda b, u: (b, 0, 0)),
            pl.BlockSpec((1, _UB, A), lambda b, u: (b, u, 0)),
            pl.BlockSpec((1, _UB, C), lambda b, u: (b, u, 0)),
            pl.BlockSpec((1, _UB, C, 1), lambda b, u: (b, u, 0, 0)),
            pl.BlockSpec((V1, A + Dh), lambda b, u: (0, 0)),
            pl.BlockSpec((Dh, J), lambda b, u: (0, 0)),
            pl.BlockSpec((1, J), lambda b, u: (0, 0)),
        ],
        out_specs=[
            pl.BlockSpec((1, T, _UB, V1), lambda b, u: (b, 0, u, 0)),
            pl.BlockSpec((1, T, _UB, Dh), lambda b, u: (b, 0, u, 0)),
            pl.BlockSpec((1, T, _UB, J), lambda b, u: (b, 0, u, 0)),
        ],
        out_shape=(
            jax.ShapeDtypeStruct((B, T, U, V1), f32),
            jax.ShapeDtypeStruct((B, T, U, Dh), f32),
            jax.ShapeDtypeStruct((B, T, U, J), f32),
        ),
    )(qac3, qse3, masks_row, masks_col, kv, Wd, bd.reshape(1, J))

    return (ptr, h_ptr, dbias)
